# baseline (speedup 1.0000x reference)
import functools
import math

import jax
import jax.numpy as jnp
from jax.experimental import pallas as pl

SPARSITY = 0.05
ROW_BLOCK = 256


def _kwta_block(x_ref, o_ref, *, k):
    x = x_ref[...]
    rows = x.shape[0]
    raw = jax.lax.bitcast_convert_type(x, jnp.int32)
    key = jnp.where(raw < 0, raw ^ jnp.int32(0x7FFFFFFF), raw)

    def count_ge(t):
        m = (key >= t).astype(jnp.int32)
        return jnp.sum(m, axis=1, keepdims=True)

    kk = jnp.int32(k)
    lo0 = jnp.full((rows, 1), jnp.int32(-2147483648))
    hi0 = jnp.full((rows, 1), jnp.int32(2147483647))

    def active(lo, hi):
        width = hi - lo
        return jnp.logical_or(width < 0, width > 1)

    def cond(state):
        lo, hi = state
        return jnp.any(active(lo, hi))

    def body(state):
        lo, hi = state
        mid = (lo >> 1) + (hi >> 1) + (lo & hi & jnp.int32(1))
        c = count_ge(mid)
        take = c >= kk
        lo = jnp.where(take, mid, lo)
        hi = jnp.where(take, hi, mid)
        hi = jnp.where(c == kk, mid + jnp.int32(1), hi)
        return lo, hi

    lo, hi = jax.lax.while_loop(cond, body, (lo0, hi0))
    o_ref[...] = jnp.where(key >= lo, x, jnp.float32(0.0))


def kernel(tensor):
    original_shape = tensor.shape
    t = tensor.reshape(tensor.shape[0] * tensor.shape[1], -1)
    n_rows, embedding_size = t.shape
    k = int(math.ceil(SPARSITY * embedding_size))
    grid = (n_rows // ROW_BLOCK,)
    out = pl.pallas_call(
        functools.partial(_kwta_block, k=k),
        grid=grid,
        in_specs=[pl.BlockSpec((ROW_BLOCK, embedding_size), lambda i: (i, 0))],
        out_specs=pl.BlockSpec((ROW_BLOCK, embedding_size), lambda i: (i, 0)),
        out_shape=jax.ShapeDtypeStruct(t.shape, t.dtype),
    )(t)
    return out.reshape(original_shape)


# speedup vs baseline: 1.2252x; 1.2252x over previous
import functools
import math

import jax
import jax.numpy as jnp
from jax.experimental import pallas as pl

SPARSITY = 0.05
ROW_BLOCK = 512


def _kwta_block(x_ref, o_ref, *, k):
    x = x_ref[...]
    raw = jax.lax.bitcast_convert_type(x, jnp.int32)
    key = jnp.where(raw < 0, raw ^ jnp.int32(0x7FFFFFFF), raw)

    def count_ge(t):
        m = (key >= t).astype(jnp.int32)
        return jnp.sum(m, axis=1, keepdims=True)

    rows = x.shape[0]
    zero = jnp.zeros((rows, 1), jnp.int32)
    int_min = jnp.full((rows, 1), jnp.int32(-2147483648))
    c0 = count_ge(zero)
    prefix = jnp.where(c0 >= k, zero, int_min)
    for b in range(30, -1, -1):
        cand = prefix + jnp.int32(1 << b)
        c = count_ge(cand)
        prefix = jnp.where(c >= k, cand, prefix)
    o_ref[...] = jnp.where(key >= prefix, x, jnp.float32(0.0))


def kernel(tensor):
    original_shape = tensor.shape
    t = tensor.reshape(tensor.shape[0] * tensor.shape[1], -1)
    n_rows, embedding_size = t.shape
    k = int(math.ceil(SPARSITY * embedding_size))
    grid = (n_rows // ROW_BLOCK,)
    out = pl.pallas_call(
        functools.partial(_kwta_block, k=k),
        grid=grid,
        in_specs=[pl.BlockSpec((ROW_BLOCK, embedding_size), lambda i: (i, 0))],
        out_specs=pl.BlockSpec((ROW_BLOCK, embedding_size), lambda i: (i, 0)),
        out_shape=jax.ShapeDtypeStruct(t.shape, t.dtype),
    )(t)
    return out.reshape(original_shape)
